# initial kernel scaffold (unmeasured)
import jax
import jax.numpy as jnp
from jax import lax
from jax.experimental import pallas as pl
from jax.experimental.pallas import tpu as pltpu


def kernel(
    x,
):
    def body(*refs):
        pass

    out_shape = jax.ShapeDtypeStruct(..., jnp.float32)
    return pl.pallas_call(body, out_shape=out_shape)(...)


# baseline (device time: 319840 ns/iter reference)
import jax
import jax.numpy as jnp
from jax import lax
from jax.experimental import pallas as pl
from jax.experimental.pallas import tpu as pltpu

N_Z = 4


def kernel(x):
    x16 = x.astype(jnp.bfloat16)
    m_per, n = x16.shape

    def body(x_ref, out_ref, comm_ref, send_sems, recv_sems, copy_sems):
        my_x = lax.axis_index("x")
        my_y = lax.axis_index("y")
        my_z = lax.axis_index("z")
        left = (my_z - 1) % N_Z
        right = (my_z + 1) % N_Z

        barrier_sem = pltpu.get_barrier_semaphore()
        for nbr in [left, right]:
            pl.semaphore_signal(
                barrier_sem,
                inc=1,
                device_id=(my_x, my_y, nbr),
                device_id_type=pl.DeviceIdType.MESH,
            )
        pl.semaphore_wait(barrier_sem, 2)

        own_copy = pltpu.make_async_copy(
            x_ref, out_ref.at[pl.ds(my_z * m_per, m_per), :], copy_sems.at[0]
        )
        own_copy.start()
        comm_ref[0] = x_ref[...]
        own_copy.wait()

        for h in range(N_Z - 1):
            send_slot = h % 2
            recv_slot = (h + 1) % 2
            rdma = pltpu.make_async_remote_copy(
                src_ref=comm_ref.at[send_slot],
                dst_ref=comm_ref.at[recv_slot],
                send_sem=send_sems.at[send_slot],
                recv_sem=recv_sems.at[recv_slot],
                device_id=(my_x, my_y, right),
                device_id_type=pl.DeviceIdType.MESH,
            )
            rdma.start()
            rdma.wait()

            origin = (my_z - h - 1) % N_Z
            chunk_copy = pltpu.make_async_copy(
                comm_ref.at[recv_slot],
                out_ref.at[pl.ds(origin * m_per, m_per), :],
                copy_sems.at[1],
            )
            chunk_copy.start()
            chunk_copy.wait()

    return pl.pallas_call(
        body,
        out_shape=jax.ShapeDtypeStruct((N_Z * m_per, n), jnp.bfloat16),
        in_specs=[pl.BlockSpec(memory_space=pltpu.VMEM)],
        out_specs=pl.BlockSpec(memory_space=pltpu.MemorySpace.HBM),
        scratch_shapes=[
            pltpu.VMEM((2, m_per, n), jnp.bfloat16),
            pltpu.SemaphoreType.DMA((2,)),
            pltpu.SemaphoreType.DMA((2,)),
            pltpu.SemaphoreType.DMA((2,)),
        ],
        compiler_params=pltpu.CompilerParams(collective_id=0),
    )(x16)


# device time: 172365 ns/iter; 1.8556x vs baseline; 1.8556x over previous
import jax
import jax.numpy as jnp
from jax import lax
from jax.experimental import pallas as pl
from jax.experimental.pallas import tpu as pltpu

N_Z = 4
MESH = pl.DeviceIdType.MESH


def kernel(x):
    x16 = x.astype(jnp.bfloat16)
    m_per, n = x16.shape
    Q = m_per // 4
    H = Q // 2

    def body(x_hbm, out_hbm, qmine, qx, qy, qdx, qdy,
             z_ssem, z_rsem, x_ssem, x_rsem, y_ssem, y_rsem,
             rx_ssem, rx_rsem, ry_ssem, ry_rsem, asm_sems, in_sems):
        my_x = lax.axis_index("x")
        my_y = lax.axis_index("y")
        my_z = lax.axis_index("z")
        qi = 2 * my_x + my_y
        qx_idx = 2 * (1 - my_x) + my_y
        qy_idx = 2 * my_x + (1 - my_y)
        qd_idx = 2 * (1 - my_x) + (1 - my_y)
        z_left = (my_z - 1) % N_Z
        z_right = (my_z + 1) % N_Z
        xp = (1 - my_x, my_y, my_z)
        yp = (my_x, 1 - my_y, my_z)

        in_q = pltpu.make_async_copy(
            x_hbm.at[pl.ds(qi * Q, Q), :], qmine.at[0], in_sems.at[0])
        in_q.start()
        own = pltpu.make_async_copy(
            x_hbm, out_hbm.at[pl.ds(my_z * m_per, m_per), :], in_sems.at[1])
        own.start()

        barrier_sem = pltpu.get_barrier_semaphore()
        for dev in [(my_x, my_y, z_left), (my_x, my_y, z_right), xp, yp]:
            pl.semaphore_signal(barrier_sem, inc=1, device_id=dev,
                                device_id_type=MESH)
        pl.semaphore_wait(barrier_sem, 4)

        in_q.wait()

        def z_fwd(r):
            return pltpu.make_async_remote_copy(
                src_ref=qmine.at[r], dst_ref=qmine.at[r + 1],
                send_sem=z_ssem.at[r], recv_sem=z_rsem.at[r],
                device_id=(my_x, my_y, z_right), device_id_type=MESH)

        pending = []
        z_rdma = z_fwd(0)
        z_rdma.start()
        pending.append(z_rdma)
        prev_asm = [own]

        for r in range(1, N_Z):
            s = r - 1
            z_rdma.wait_recv()
            if r < N_Z - 1:
                z_rdma = z_fwd(r)
                z_rdma.start()
                pending.append(z_rdma)

            px = pltpu.make_async_remote_copy(
                src_ref=qmine.at[r], dst_ref=qx.at[s],
                send_sem=x_ssem.at[s], recv_sem=x_rsem.at[s],
                device_id=xp, device_id_type=MESH)
            py = pltpu.make_async_remote_copy(
                src_ref=qmine.at[r], dst_ref=qy.at[s],
                send_sem=y_ssem.at[s], recv_sem=y_rsem.at[s],
                device_id=yp, device_id_type=MESH)
            px.start()
            py.start()
            pending += [px, py]

            px.wait_recv()
            ry = pltpu.make_async_remote_copy(
                src_ref=qx.at[s, pl.ds(H, H), :], dst_ref=qdy.at[s],
                send_sem=ry_ssem.at[s], recv_sem=ry_rsem.at[s],
                device_id=yp, device_id_type=MESH)
            ry.start()
            pending.append(ry)
            py.wait_recv()
            rx = pltpu.make_async_remote_copy(
                src_ref=qy.at[s, pl.ds(0, H), :], dst_ref=qdx.at[s],
                send_sem=rx_ssem.at[s], recv_sem=rx_rsem.at[s],
                device_id=xp, device_id_type=MESH)
            rx.start()
            pending.append(rx)
            rx.wait_recv()
            ry.wait_recv()

            for c in prev_asm:
                c.wait()
            base = ((my_z - r) % N_Z) * m_per
            prev_asm = []
            for src, row0, nrows, sem in [
                (qmine.at[r], base + qi * Q, Q, 0),
                (qx.at[s], base + qx_idx * Q, Q, 1),
                (qy.at[s], base + qy_idx * Q, Q, 2),
                (qdx.at[s], base + qd_idx * Q, H, 3),
                (qdy.at[s], base + qd_idx * Q + H, H, 4),
            ]:
                cp = pltpu.make_async_copy(
                    src, out_hbm.at[pl.ds(row0, nrows), :], asm_sems.at[sem])
                cp.start()
                prev_asm.append(cp)

        for c in prev_asm:
            c.wait()
        for d in pending:
            d.wait_send()

    return pl.pallas_call(
        body,
        out_shape=jax.ShapeDtypeStruct((N_Z * m_per, n), jnp.bfloat16),
        in_specs=[pl.BlockSpec(memory_space=pltpu.MemorySpace.HBM)],
        out_specs=pl.BlockSpec(memory_space=pltpu.MemorySpace.HBM),
        scratch_shapes=[
            pltpu.VMEM((N_Z, Q, n), jnp.bfloat16),
            pltpu.VMEM((N_Z - 1, Q, n), jnp.bfloat16),
            pltpu.VMEM((N_Z - 1, Q, n), jnp.bfloat16),
            pltpu.VMEM((N_Z - 1, H, n), jnp.bfloat16),
            pltpu.VMEM((N_Z - 1, H, n), jnp.bfloat16),
            pltpu.SemaphoreType.DMA((N_Z - 1,)),
            pltpu.SemaphoreType.DMA((N_Z - 1,)),
            pltpu.SemaphoreType.DMA((N_Z - 1,)),
            pltpu.SemaphoreType.DMA((N_Z - 1,)),
            pltpu.SemaphoreType.DMA((N_Z - 1,)),
            pltpu.SemaphoreType.DMA((N_Z - 1,)),
            pltpu.SemaphoreType.DMA((N_Z - 1,)),
            pltpu.SemaphoreType.DMA((N_Z - 1,)),
            pltpu.SemaphoreType.DMA((N_Z - 1,)),
            pltpu.SemaphoreType.DMA((N_Z - 1,)),
            pltpu.SemaphoreType.DMA((5,)),
            pltpu.SemaphoreType.DMA((2,)),
        ],
        compiler_params=pltpu.CompilerParams(collective_id=0),
    )(x16)


# device time: 106431 ns/iter; 3.0051x vs baseline; 1.6195x over previous
import jax
import jax.numpy as jnp
from jax import lax
from jax.experimental import pallas as pl
from jax.experimental.pallas import tpu as pltpu

N_Z = 4
MESH = pl.DeviceIdType.MESH


def kernel(x):
    x16 = x.astype(jnp.bfloat16)
    m_per, n = x16.shape
    Q = m_per // 4
    H = Q // 2

    def body(x_hbm, out_hbm, qmine, qx, qy, qdx, qdy,
             z_ssem, z_rsem, x_ssem, x_rsem, y_ssem, y_rsem,
             rx_ssem, rx_rsem, ry_ssem, ry_rsem, asm_sems, in_sems):
        my_x = lax.axis_index("x")
        my_y = lax.axis_index("y")
        my_z = lax.axis_index("z")
        qi = 2 * my_x + my_y
        qx_idx = 2 * (1 - my_x) + my_y
        qy_idx = 2 * my_x + (1 - my_y)
        qd_idx = 2 * (1 - my_x) + (1 - my_y)
        z_left = (my_z - 1) % N_Z
        z_right = (my_z + 1) % N_Z
        xp = (1 - my_x, my_y, my_z)
        yp = (my_x, 1 - my_y, my_z)

        in_q = pltpu.make_async_copy(
            x_hbm.at[pl.ds(qi * Q, Q), :], qmine.at[0], in_sems.at[0])
        in_q.start()
        own = pltpu.make_async_copy(
            x_hbm, out_hbm.at[pl.ds(my_z * m_per, m_per), :], in_sems.at[1])
        own.start()

        barrier_sem = pltpu.get_barrier_semaphore()
        for dev in [(my_x, my_y, z_left), (my_x, my_y, z_right), xp, yp]:
            pl.semaphore_signal(barrier_sem, inc=1, device_id=dev,
                                device_id_type=MESH)
        pl.semaphore_wait(barrier_sem, 4)

        in_q.wait()

        def z_fwd(r):
            return pltpu.make_async_remote_copy(
                src_ref=qmine.at[r], dst_ref=qmine.at[r + 1],
                send_sem=z_ssem.at[r], recv_sem=z_rsem.at[r],
                device_id=(my_x, my_y, z_right), device_id_type=MESH)

        pending = []
        z_rdma = z_fwd(0)
        z_rdma.start()
        pending.append(z_rdma)
        prev_asm = [own]

        def finish_round(s, px, py):
            px.wait_recv()
            ry = pltpu.make_async_remote_copy(
                src_ref=qx.at[s, pl.ds(H, H), :], dst_ref=qdy.at[s],
                send_sem=ry_ssem.at[s], recv_sem=ry_rsem.at[s],
                device_id=yp, device_id_type=MESH)
            ry.start()
            pending.append(ry)
            py.wait_recv()
            rx = pltpu.make_async_remote_copy(
                src_ref=qy.at[s, pl.ds(0, H), :], dst_ref=qdx.at[s],
                send_sem=rx_ssem.at[s], recv_sem=rx_rsem.at[s],
                device_id=xp, device_id_type=MESH)
            rx.start()
            pending.append(rx)
            rx.wait_recv()
            ry.wait_recv()

            for c in prev_asm:
                c.wait()
            prev_asm.clear()
            base = ((my_z - (s + 1)) % N_Z) * m_per
            for src, row0, nrows, sem in [
                (qmine.at[s + 1], base + qi * Q, Q, 0),
                (qx.at[s], base + qx_idx * Q, Q, 1),
                (qy.at[s], base + qy_idx * Q, Q, 2),
                (qdx.at[s], base + qd_idx * Q, H, 3),
                (qdy.at[s], base + qd_idx * Q + H, H, 4),
            ]:
                cp = pltpu.make_async_copy(
                    src, out_hbm.at[pl.ds(row0, nrows), :], asm_sems.at[sem])
                cp.start()
                prev_asm.append(cp)

        plane_prev = None
        for r in range(1, N_Z):
            s = r - 1
            z_rdma.wait_recv()
            if r < N_Z - 1:
                z_rdma = z_fwd(r)
                z_rdma.start()
                pending.append(z_rdma)

            px = pltpu.make_async_remote_copy(
                src_ref=qmine.at[r], dst_ref=qx.at[s],
                send_sem=x_ssem.at[s], recv_sem=x_rsem.at[s],
                device_id=xp, device_id_type=MESH)
            py = pltpu.make_async_remote_copy(
                src_ref=qmine.at[r], dst_ref=qy.at[s],
                send_sem=y_ssem.at[s], recv_sem=y_rsem.at[s],
                device_id=yp, device_id_type=MESH)
            px.start()
            py.start()
            pending += [px, py]

            if plane_prev is not None:
                finish_round(*plane_prev)
            plane_prev = (s, px, py)

        finish_round(*plane_prev)
        for c in prev_asm:
            c.wait()
        for d in pending:
            d.wait_send()

    return pl.pallas_call(
        body,
        out_shape=jax.ShapeDtypeStruct((N_Z * m_per, n), jnp.bfloat16),
        in_specs=[pl.BlockSpec(memory_space=pltpu.MemorySpace.HBM)],
        out_specs=pl.BlockSpec(memory_space=pltpu.MemorySpace.HBM),
        scratch_shapes=[
            pltpu.VMEM((N_Z, Q, n), jnp.bfloat16),
            pltpu.VMEM((N_Z - 1, Q, n), jnp.bfloat16),
            pltpu.VMEM((N_Z - 1, Q, n), jnp.bfloat16),
            pltpu.VMEM((N_Z - 1, H, n), jnp.bfloat16),
            pltpu.VMEM((N_Z - 1, H, n), jnp.bfloat16),
            pltpu.SemaphoreType.DMA((N_Z - 1,)),
            pltpu.SemaphoreType.DMA((N_Z - 1,)),
            pltpu.SemaphoreType.DMA((N_Z - 1,)),
            pltpu.SemaphoreType.DMA((N_Z - 1,)),
            pltpu.SemaphoreType.DMA((N_Z - 1,)),
            pltpu.SemaphoreType.DMA((N_Z - 1,)),
            pltpu.SemaphoreType.DMA((N_Z - 1,)),
            pltpu.SemaphoreType.DMA((N_Z - 1,)),
            pltpu.SemaphoreType.DMA((N_Z - 1,)),
            pltpu.SemaphoreType.DMA((N_Z - 1,)),
            pltpu.SemaphoreType.DMA((5,)),
            pltpu.SemaphoreType.DMA((2,)),
        ],
        compiler_params=pltpu.CompilerParams(collective_id=0),
    )(x16)
